# baseline (speedup 1.0000x reference)
import jax
import jax.numpy as jnp
from jax import lax
from jax.experimental import pallas as pl
from jax.experimental.pallas import tpu as pltpu
from jax.experimental.pallas import tpu_sc as plsc

_S = 24
_N = 577
_D = 64
_NV = _D // 16
_NC = 2
_NS = 16
_NW = _NC * _NS
_RPW = 19
_HA = 288


def _rp2d_body(v_hbm, h_hbm, out_hbm, v_vm, h_vm, row_vm, sem_a, sem_b):
    w = lax.axis_index("s") * _NC + lax.axis_index("c")
    pltpu.sync_copy(v_hbm, v_vm)
    pltpu.sync_copy(h_hbm, h_vm)

    cls_vec = [v_vm[0, pl.ds(d * 16, 16)] + h_vm[0, pl.ds(d * 16, 16)]
               for d in range(_NV)]

    def wait_half(sem, lo, n):
        pltpu.make_async_copy(
            row_vm.at[pl.ds(lo, n)], out_hbm.at[0, pl.ds(lo, n)], sem).wait()

    def _slice_starts(q):
        qb = (q - 1) // _S
        qr = (q - 1) % _S
        return (_S + 1) - qb, (_S + 1) - qr

    def _emit_group(vb, hb, kbs, last_kr_hi=_S):
        vv = [[v_vm[vb + kb, pl.ds(d * 16, 16)] for d in range(_NV)]
              for kb in kbs]
        for kr in range(_S):
            hrow = hb + kr
            for d in range(_NV):
                h = h_vm[hrow, pl.ds(d * 16, 16)]
                for i, kb in enumerate(kbs):
                    if i == len(kbs) - 1 and kr >= last_kr_hi:
                        continue
                    r = 1 + kb * _S + kr
                    row_vm[r, pl.ds(d * 16, 16)] = vv[i][d] + h

    def do_row(j, carry):
        q = w + _NW * j

        @pl.when(q < _N)
        def _():
            @pl.when(j >= 1)
            def _():
                wait_half(sem_a, 0, _HA)

            @pl.when(q == 0)
            def _():
                def fill(k, c):
                    for d in range(_NV):
                        row_vm[k, pl.ds(d * 16, 16)] = cls_vec[d]
                    return c
                lax.fori_loop(0, _N, fill, 0)

            @pl.when(q > 0)
            def _():
                vb, hb = _slice_starts(q)
                for d in range(_NV):
                    row_vm[0, pl.ds(d * 16, 16)] = cls_vec[d]

                @plsc.parallel_loop(0, 2, 1, unroll=1)
                def _(g):
                    kb0 = g * 4
                    _emit_group(vb, hb, [kb0, kb0 + 1, kb0 + 2, kb0 + 3])

                _emit_group(vb, hb, [8, 9, 10, 11], last_kr_hi=_S - 1)

            pltpu.async_copy(row_vm.at[pl.ds(0, _HA)],
                             out_hbm.at[q, pl.ds(0, _HA)], sem_a)

            @pl.when(j >= 1)
            def _():
                wait_half(sem_b, _HA, _N - _HA)

            @pl.when(q > 0)
            def _():
                vb, hb = _slice_starts(q)
                vv11 = [v_vm[vb + 11, pl.ds(d * 16, 16)]
                        for d in range(_NV)]
                for d in range(_NV):
                    row_vm[_HA, pl.ds(d * 16, 16)] = (
                        vv11[d] + h_vm[hb + _S - 1, pl.ds(d * 16, 16)])

                @plsc.parallel_loop(3, 6, 1, unroll=1)
                def _(g):
                    kb0 = g * 4
                    _emit_group(vb, hb, [kb0, kb0 + 1, kb0 + 2, kb0 + 3])

            pltpu.async_copy(row_vm.at[pl.ds(_HA, _N - _HA)],
                             out_hbm.at[q, pl.ds(_HA, _N - _HA)], sem_b)

        return carry

    lax.fori_loop(0, _RPW, do_row, 0)
    wait_half(sem_a, 0, _HA)
    wait_half(sem_b, _HA, _N - _HA)


@jax.jit
def _rp2d(table_v, table_h):
    mesh = plsc.VectorSubcoreMesh(
        core_axis_name="c", subcore_axis_name="s",
        num_cores=_NC, num_subcores=_NS)
    return pl.kernel(
        _rp2d_body,
        out_type=jax.ShapeDtypeStruct((_N, _N, _D), jnp.float32),
        mesh=mesh,
        scratch_types=[
            pltpu.VMEM((2 * _S + 2, _D), jnp.float32),
            pltpu.VMEM((2 * _S + 2, _D), jnp.float32),
            pltpu.VMEM((_N, _D), jnp.float32),
            pltpu.SemaphoreType.DMA,
            pltpu.SemaphoreType.DMA,
        ],
    )(table_v, table_h)


def kernel(length_q, length_k, embeddings_table_v, embeddings_table_h):
    del length_q, length_k
    return _rp2d(embeddings_table_v, embeddings_table_h)
